# baseline (speedup 1.0000x reference)
import jax
import jax.numpy as jnp
from jax import lax
from jax.experimental import pallas as pl
from jax.experimental.pallas import tpu as pltpu
from jax.experimental.pallas import tpu_sc as plsc

N = 50000
E = 150000
H = 128

BN = 1000
GRID = N // BN

NT = 16
CHUNK = 128
NP = 74
E_PAD = NT * NP * CHUNK
QROWS = 12544
ACC_TOT = 12672
LDUMMY = QROWS
ZPT = ACC_TOT // NT
DZ = 24
DPT = QROWS // NT
DD = 56
AGG_ROWS = 4 * QROWS
DUMMY = N


def _relu(x):
    return jnp.maximum(x, 0.0)


def _dot(a, b):
    return jnp.dot(a, b, preferred_element_type=jnp.float32)


def _enc_body(xb_r, xj_r, xf_r, Wb_r, bb_r, Wj_r, bj_r, Wf_r, bf_r,
              ob_d, oj_d, of_d):
    ob_d[...] = _relu(_dot(xb_r[...], Wb_r[...]) + bb_r[...])
    oj_d[...] = _relu(_dot(xj_r[...], Wj_r[...]) + bj_r[...])
    of_d[...] = _relu(_dot(xf_r[...], Wf_r[...]) + bf_r[...])


def _encoder(x_base, x_joint, x_foot, Wb, bb, Wj, bj, Wf, bf):
    row = pl.BlockSpec((BN, H), lambda i: (i, 0))
    w = pl.BlockSpec((H, H), lambda i: (0, 0))
    b = pl.BlockSpec((1, H), lambda i: (0, 0))
    x = jax.ShapeDtypeStruct((N, H), jnp.float32)
    return pl.pallas_call(
        _enc_body,
        grid=(GRID,),
        in_specs=[row, row, row, w, b, w, b, w, b],
        out_specs=(row, row, row),
        out_shape=(x, x, x),
    )(x_base, x_joint, x_foot, Wb, bb, Wj, bj, Wf, bf)


def _layer0_body(xj_r, xf_r, abj_r, ajf_r,
                 Wr2_r, Wro2_r, b2_r, Wr3_r, Wro3_r, b3_r,
                 oj_d, of_d):
    oj = _dot(abj_r[...], Wr2_r[...]) + _dot(xj_r[...], Wro2_r[...]) + b2_r[...]
    oj_d[...] = xj_r[...] + _relu(oj)
    of = _dot(ajf_r[...], Wr3_r[...]) + _dot(xf_r[...], Wro3_r[...]) + b3_r[...]
    of_d[...] = xf_r[...] + _relu(of)


def _layer0(xj0, xf0, agg_bj, agg_jf0, Wr2, Wro2, b2, Wr3, Wro3, b3):
    row = pl.BlockSpec((BN, H), lambda i: (i, 0))
    w = pl.BlockSpec((H, H), lambda i: (0, 0))
    b = pl.BlockSpec((1, H), lambda i: (0, 0))
    x = jax.ShapeDtypeStruct((N, H), jnp.float32)
    return pl.pallas_call(
        _layer0_body,
        grid=(GRID,),
        in_specs=[row, row, row, row, w, w, b, w, w, b],
        out_specs=(row, row),
        out_shape=(x, x),
    )(xj0, xf0, agg_bj, agg_jf0, Wr2, Wro2, b2, Wr3, Wro3, b3)


def _layer1_body(xf_r, ajf_r, Wr3_r, Wro3_r, b3_r, Wd_r, bd_r, y_d):
    of = _dot(ajf_r[...], Wr3_r[...]) + _dot(xf_r[...], Wro3_r[...]) + b3_r[...]
    xf2 = xf_r[...] + _relu(of)
    y_d[...] = _dot(xf2, Wd_r[...]) + bd_r[...]


def _layer1(xf1, agg_jf1, Wr3, Wro3, b3, Wd, bd):
    row = pl.BlockSpec((BN, H), lambda i: (i, 0))
    w = pl.BlockSpec((H, H), lambda i: (0, 0))
    b = pl.BlockSpec((1, H), lambda i: (0, 0))
    return pl.pallas_call(
        _layer1_body,
        grid=(GRID,),
        in_specs=[row, row, w, w, b,
                  pl.BlockSpec((H, 1), lambda i: (0, 0)),
                  pl.BlockSpec((1, 1), lambda i: (0, 0))],
        out_specs=pl.BlockSpec((BN, 1), lambda i: (i, 0)),
        out_shape=jax.ShapeDtypeStruct((N, 1), jnp.float32),
    )(xf1, agg_jf1, Wr3, Wro3, b3, Wd, bd)


def _segsum_sc(ops, zeros_dz):
    nops = len(ops)
    mesh = plsc.VectorSubcoreMesh(core_axis_name="c", subcore_axis_name="s")

    def body(*refs):
        ins = refs[:3 * nops + 1]
        outs = refs[3 * nops + 1: 4 * nops + 1]
        (sidx, didx, ldst, gbufA, gbufB, zbuf, dbuf, acc,
         sem_gA, sem_gB, sem_i) = refs[4 * nops + 1:]
        cid = lax.axis_index("c")
        sid = lax.axis_index("s")
        zeros_hbm = ins[3 * nops]
        pltpu.sync_copy(zeros_hbm, zbuf)

        for op in range(nops):
            x_hbm = ins[3 * op]
            src_h = ins[3 * op + 1]
            dst_h = ins[3 * op + 2]
            agg = outs[op]
            for half in range(2):
                qlo = (cid * 2 + half) * QROWS

                def remap(p):
                    for j2 in range(2):
                        for j in range(4):
                            d = didx[p, pl.ds(j2 * 64 + j * 16, 16)]
                            ok = (d >= qlo) & (d < qlo + QROWS)
                            ldst[p, j2, pl.ds(j * 16, 16)] = jnp.where(
                                ok, d - qlo, LDUMMY)

                def idx_load(pair, p):
                    pltpu.async_copy(
                        src_h.at[sid, pl.ds(pair, 1)],
                        sidx.at[pl.ds(p, 1)], sem_i)
                    pltpu.async_copy(
                        dst_h.at[sid, pl.ds(pair, 1)],
                        didx.at[pl.ds(p, 1)], sem_i)

                def idx_wait(p):
                    for r in (sidx, didx):
                        pltpu.make_async_copy(
                            src_h.at[sid, pl.ds(0, 1)],
                            r.at[pl.ds(p, 1)], sem_i).wait()

                def gather(p, j2, gbuf, sem):
                    pltpu.async_copy(
                        x_hbm.at[sidx.at[p, pl.ds(j2 * 64, 64)]], gbuf, sem)

                def gwait(gbuf, sem):
                    pltpu.make_async_copy(
                        x_hbm.at[sidx.at[0, pl.ds(0, 64)]], gbuf, sem).wait()

                def zbody(i, _):
                    pltpu.sync_copy(zbuf, acc.at[pl.ds(sid * ZPT + i * DZ, DZ)])
                    return 0

                lax.fori_loop(0, ZPT // DZ, zbody, 0)
                plsc.subcore_barrier()

                idx_load(0, 0)
                idx_wait(0)
                remap(0)
                idx_load(1, 1)
                pltpu.async_copy(
                    x_hbm.at[sidx.at[0, pl.ds(0, 64)]], gbufA, sem_gA)
                pltpu.async_copy(
                    x_hbm.at[sidx.at[0, pl.ds(64, 64)]], gbufB, sem_gB)

                def pair_body(k, P, last):
                    Pn = 1 - P
                    idx_wait(Pn)
                    remap(Pn)
                    gwait(gbufA, sem_gA)
                    pltpu.sync_copy(gbufA, acc.at[ldst.at[P, 0]], add=True)
                    if not last:
                        gather(Pn, 0, gbufA, sem_gA)
                    gwait(gbufB, sem_gB)
                    pltpu.sync_copy(gbufB, acc.at[ldst.at[P, 1]], add=True)
                    if not last:
                        gather(Pn, 1, gbufB, sem_gB)
                    idx_load(jnp.minimum(k + 2, NP - 1), P)

                def kkbody(kk, _):
                    @pl.when(kk < (NP // 2) - 1)
                    def _():
                        pair_body(2 * kk, 0, False)
                        pair_body(2 * kk + 1, 1, False)

                    @pl.when(kk == (NP // 2) - 1)
                    def _():
                        pair_body(2 * kk, 0, False)
                        pair_body(2 * kk + 1, 1, True)

                    return 0

                lax.fori_loop(0, NP // 2, kkbody, 0)
                idx_wait(0)
                plsc.subcore_barrier()

                def dbody(i, _):
                    r = sid * DPT + i * DD
                    pltpu.sync_copy(acc.at[pl.ds(r, DD)], dbuf)
                    pltpu.sync_copy(
                        dbuf, agg.at[pl.ds(pl.multiple_of(qlo + r, 8), DD)])
                    return 0

                lax.fori_loop(0, DPT // DD, dbody, 0)
                plsc.subcore_barrier()

    f = pl.kernel(
        body,
        out_type=tuple(jax.ShapeDtypeStruct((AGG_ROWS, H), jnp.float32)
                       for _ in range(nops)),
        mesh=mesh,
        scratch_types=[
            pltpu.VMEM((2, CHUNK), jnp.int32),
            pltpu.VMEM((2, CHUNK), jnp.int32),
            pltpu.VMEM((2, 2, 64), jnp.int32),
            pltpu.VMEM((64, H), jnp.float32),
            pltpu.VMEM((64, H), jnp.float32),
            pltpu.VMEM((DZ, H), jnp.float32),
            pltpu.VMEM((DD, H), jnp.float32),
            pltpu.VMEM_SHARED((ACC_TOT, H), jnp.float32),
            pltpu.SemaphoreType.DMA,
            pltpu.SemaphoreType.DMA,
            pltpu.SemaphoreType.DMA,
        ],
    )
    args = []
    for (x, s, d) in ops:
        args += [x, s, d]
    args.append(zeros_dz)
    return f(*args)


def _prep_edges(ei):
    src = jnp.concatenate([ei[0], jnp.zeros((E_PAD - E,), jnp.int32)])
    dst = jnp.concatenate([ei[1], jnp.full((E_PAD - E,), DUMMY, jnp.int32)])
    return src.reshape(NT, NP, CHUNK), dst.reshape(NT, NP, CHUNK)


def kernel(x_base, x_joint, x_foot, ei_gt, ei_gs, ei_bj, ei_jf,
           W_enc_base, b_enc_base, W_enc_joint, b_enc_joint,
           W_enc_foot, b_enc_foot, W_rel, b_rel, W_root,
           W_bt1, b_bt1, W_bt2, b_bt2, W_dec, b_dec):
    sbj, dbj = _prep_edges(ei_bj)
    sjf, djf = _prep_edges(ei_jf)
    zeros_dz = jnp.zeros((DZ, H), jnp.float32)

    xb0, xj0, xf0 = _encoder(
        x_base, x_joint, x_foot,
        W_enc_base, b_enc_base.reshape(1, H),
        W_enc_joint, b_enc_joint.reshape(1, H),
        W_enc_foot, b_enc_foot.reshape(1, H))

    agg_bj, agg_jf0 = _segsum_sc(
        [(xb0, sbj, dbj), (xj0, sjf, djf)], zeros_dz)

    xj1, xf1 = _layer0(
        xj0, xf0, agg_bj, agg_jf0,
        W_rel[0, 2], W_root[0, 2], b_rel[0, 2].reshape(1, H),
        W_rel[0, 3], W_root[0, 3], b_rel[0, 3].reshape(1, H))

    (agg_jf1,) = _segsum_sc([(xj1, sjf, djf)], zeros_dz)

    y = _layer1(
        xf1, agg_jf1,
        W_rel[1, 3], W_root[1, 3], b_rel[1, 3].reshape(1, H),
        W_dec, b_dec.reshape(1, 1))
    return y


# speedup vs baseline: 1.0052x; 1.0052x over previous
import jax
import jax.numpy as jnp
from jax import lax
from jax.experimental import pallas as pl
from jax.experimental.pallas import tpu as pltpu
from jax.experimental.pallas import tpu_sc as plsc

N = 50000
E = 150000
H = 128

BN = 1000
GRID = N // BN

NT = 16
CHUNK = 128
NP = 74
E_PAD = NT * NP * CHUNK
QROWS = 12544
ACC_TOT = 12672
LDUMMY = QROWS
ZPT = ACC_TOT // NT
DZ = 24
DPT = QROWS // NT
DD = 56
AGG_ROWS = 4 * QROWS
DUMMY = N


def _relu(x):
    return jnp.maximum(x, 0.0)


def _dot(a, b):
    return jnp.dot(a, b, preferred_element_type=jnp.float32)


def _enc_body(xb_r, xj_r, xf_r, Wb_r, bb_r, Wj_r, bj_r, Wf_r, bf_r,
              ob_d, oj_d, of_d):
    ob_d[...] = _relu(_dot(xb_r[...], Wb_r[...]) + bb_r[...])
    oj_d[...] = _relu(_dot(xj_r[...], Wj_r[...]) + bj_r[...])
    of_d[...] = _relu(_dot(xf_r[...], Wf_r[...]) + bf_r[...])


def _encoder(x_base, x_joint, x_foot, Wb, bb, Wj, bj, Wf, bf):
    row = pl.BlockSpec((BN, H), lambda i: (i, 0))
    w = pl.BlockSpec((H, H), lambda i: (0, 0))
    b = pl.BlockSpec((1, H), lambda i: (0, 0))
    x = jax.ShapeDtypeStruct((N, H), jnp.float32)
    return pl.pallas_call(
        _enc_body,
        grid=(GRID,),
        in_specs=[row, row, row, w, b, w, b, w, b],
        out_specs=(row, row, row),
        out_shape=(x, x, x),
    )(x_base, x_joint, x_foot, Wb, bb, Wj, bj, Wf, bf)


def _layer0_body(xj_r, xf_r, abj_r, ajf_r,
                 Wr2_r, Wro2_r, b2_r, Wr3_r, Wro3_r, b3_r,
                 oj_d, of_d):
    oj = _dot(abj_r[...], Wr2_r[...]) + _dot(xj_r[...], Wro2_r[...]) + b2_r[...]
    oj_d[...] = xj_r[...] + _relu(oj)
    of = _dot(ajf_r[...], Wr3_r[...]) + _dot(xf_r[...], Wro3_r[...]) + b3_r[...]
    of_d[...] = xf_r[...] + _relu(of)


def _layer0(xj0, xf0, agg_bj, agg_jf0, Wr2, Wro2, b2, Wr3, Wro3, b3):
    row = pl.BlockSpec((BN, H), lambda i: (i, 0))
    w = pl.BlockSpec((H, H), lambda i: (0, 0))
    b = pl.BlockSpec((1, H), lambda i: (0, 0))
    x = jax.ShapeDtypeStruct((N, H), jnp.float32)
    return pl.pallas_call(
        _layer0_body,
        grid=(GRID,),
        in_specs=[row, row, row, row, w, w, b, w, w, b],
        out_specs=(row, row),
        out_shape=(x, x),
    )(xj0, xf0, agg_bj, agg_jf0, Wr2, Wro2, b2, Wr3, Wro3, b3)


def _layer1_body(xf_r, ajf_r, Wr3_r, Wro3_r, b3_r, Wd_r, bd_r, y_d):
    of = _dot(ajf_r[...], Wr3_r[...]) + _dot(xf_r[...], Wro3_r[...]) + b3_r[...]
    xf2 = xf_r[...] + _relu(of)
    y_d[...] = _dot(xf2, Wd_r[...]) + bd_r[...]


def _layer1(xf1, agg_jf1, Wr3, Wro3, b3, Wd, bd):
    row = pl.BlockSpec((BN, H), lambda i: (i, 0))
    w = pl.BlockSpec((H, H), lambda i: (0, 0))
    b = pl.BlockSpec((1, H), lambda i: (0, 0))
    return pl.pallas_call(
        _layer1_body,
        grid=(GRID,),
        in_specs=[row, row, w, w, b,
                  pl.BlockSpec((H, 1), lambda i: (0, 0)),
                  pl.BlockSpec((1, 1), lambda i: (0, 0))],
        out_specs=pl.BlockSpec((BN, 1), lambda i: (i, 0)),
        out_shape=jax.ShapeDtypeStruct((N, 1), jnp.float32),
    )(xf1, agg_jf1, Wr3, Wro3, b3, Wd, bd)


def _segsum_sc(ops, zeros_dz):
    nops = len(ops)
    mesh = plsc.VectorSubcoreMesh(core_axis_name="c", subcore_axis_name="s")

    def body(*refs):
        ins = refs[:3 * nops + 1]
        outs = refs[3 * nops + 1: 4 * nops + 1]
        (sidx, didx, ldst, gbufA, gbufB, zbuf, dbuf, acc,
         sem_gA, sem_gB, sem_i, sem_z) = refs[4 * nops + 1:]
        cid = lax.axis_index("c")
        sid = lax.axis_index("s")
        zeros_hbm = ins[3 * nops]
        pltpu.sync_copy(zeros_hbm, zbuf)

        for op in range(nops):
            x_hbm = ins[3 * op]
            src_h = ins[3 * op + 1]
            dst_h = ins[3 * op + 2]
            agg = outs[op]
            for half in range(2):
                qlo = (cid * 2 + half) * QROWS

                def remap(p):
                    for j2 in range(2):
                        for j in range(4):
                            d = didx[p, pl.ds(j2 * 64 + j * 16, 16)]
                            ok = (d >= qlo) & (d < qlo + QROWS)
                            ldst[p, j2, pl.ds(j * 16, 16)] = jnp.where(
                                ok, d - qlo, LDUMMY)

                def idx_load(pair, p):
                    pltpu.async_copy(
                        src_h.at[sid, pl.ds(pair, 1)],
                        sidx.at[pl.ds(p, 1)], sem_i)
                    pltpu.async_copy(
                        dst_h.at[sid, pl.ds(pair, 1)],
                        didx.at[pl.ds(p, 1)], sem_i)

                def idx_wait(p):
                    for r in (sidx, didx):
                        pltpu.make_async_copy(
                            src_h.at[sid, pl.ds(0, 1)],
                            r.at[pl.ds(p, 1)], sem_i).wait()

                def gather(p, j2, gbuf, sem):
                    pltpu.async_copy(
                        x_hbm.at[sidx.at[p, pl.ds(j2 * 64, 64)]], gbuf, sem)

                def gwait(gbuf, sem):
                    pltpu.make_async_copy(
                        x_hbm.at[sidx.at[0, pl.ds(0, 64)]], gbuf, sem).wait()

                def zbody(i, _):
                    pltpu.async_copy(
                        zbuf, acc.at[pl.ds(sid * ZPT + i * DZ, DZ)], sem_z)
                    return 0

                lax.fori_loop(0, ZPT // DZ, zbody, 0)

                def zdrain(i, _):
                    pltpu.make_async_copy(
                        zbuf, acc.at[pl.ds(sid * ZPT, DZ)], sem_z).wait()
                    return 0

                lax.fori_loop(0, ZPT // DZ, zdrain, 0)
                plsc.subcore_barrier()

                idx_load(0, 0)
                idx_wait(0)
                remap(0)
                idx_load(1, 1)
                pltpu.async_copy(
                    x_hbm.at[sidx.at[0, pl.ds(0, 64)]], gbufA, sem_gA)
                pltpu.async_copy(
                    x_hbm.at[sidx.at[0, pl.ds(64, 64)]], gbufB, sem_gB)

                def pair_body(k, P, last):
                    Pn = 1 - P
                    idx_wait(Pn)
                    remap(Pn)
                    gwait(gbufA, sem_gA)
                    pltpu.sync_copy(gbufA, acc.at[ldst.at[P, 0]], add=True)
                    if not last:
                        gather(Pn, 0, gbufA, sem_gA)
                    gwait(gbufB, sem_gB)
                    pltpu.sync_copy(gbufB, acc.at[ldst.at[P, 1]], add=True)
                    if not last:
                        gather(Pn, 1, gbufB, sem_gB)
                    idx_load(jnp.minimum(k + 2, NP - 1), P)

                def kkbody(kk, _):
                    @pl.when(kk < (NP // 2) - 1)
                    def _():
                        pair_body(2 * kk, 0, False)
                        pair_body(2 * kk + 1, 1, False)

                    @pl.when(kk == (NP // 2) - 1)
                    def _():
                        pair_body(2 * kk, 0, False)
                        pair_body(2 * kk + 1, 1, True)

                    return 0

                lax.fori_loop(0, NP // 2, kkbody, 0)
                idx_wait(0)
                plsc.subcore_barrier()

                def dbody(i, _):
                    r = sid * DPT + i * DD
                    pltpu.sync_copy(acc.at[pl.ds(r, DD)], dbuf)
                    pltpu.sync_copy(
                        dbuf, agg.at[pl.ds(pl.multiple_of(qlo + r, 8), DD)])
                    return 0

                lax.fori_loop(0, DPT // DD, dbody, 0)
                plsc.subcore_barrier()

    f = pl.kernel(
        body,
        out_type=tuple(jax.ShapeDtypeStruct((AGG_ROWS, H), jnp.float32)
                       for _ in range(nops)),
        mesh=mesh,
        scratch_types=[
            pltpu.VMEM((2, CHUNK), jnp.int32),
            pltpu.VMEM((2, CHUNK), jnp.int32),
            pltpu.VMEM((2, 2, 64), jnp.int32),
            pltpu.VMEM((64, H), jnp.float32),
            pltpu.VMEM((64, H), jnp.float32),
            pltpu.VMEM((DZ, H), jnp.float32),
            pltpu.VMEM((DD, H), jnp.float32),
            pltpu.VMEM_SHARED((ACC_TOT, H), jnp.float32),
            pltpu.SemaphoreType.DMA,
            pltpu.SemaphoreType.DMA,
            pltpu.SemaphoreType.DMA,
            pltpu.SemaphoreType.DMA,
        ],
    )
    args = []
    for (x, s, d) in ops:
        args += [x, s, d]
    args.append(zeros_dz)
    return f(*args)


def _prep_edges(ei):
    src = jnp.concatenate([ei[0], jnp.zeros((E_PAD - E,), jnp.int32)])
    dst = jnp.concatenate([ei[1], jnp.full((E_PAD - E,), DUMMY, jnp.int32)])
    return src.reshape(NT, NP, CHUNK), dst.reshape(NT, NP, CHUNK)


def kernel(x_base, x_joint, x_foot, ei_gt, ei_gs, ei_bj, ei_jf,
           W_enc_base, b_enc_base, W_enc_joint, b_enc_joint,
           W_enc_foot, b_enc_foot, W_rel, b_rel, W_root,
           W_bt1, b_bt1, W_bt2, b_bt2, W_dec, b_dec):
    sbj, dbj = _prep_edges(ei_bj)
    sjf, djf = _prep_edges(ei_jf)
    zeros_dz = jnp.zeros((DZ, H), jnp.float32)

    xb0, xj0, xf0 = _encoder(
        x_base, x_joint, x_foot,
        W_enc_base, b_enc_base.reshape(1, H),
        W_enc_joint, b_enc_joint.reshape(1, H),
        W_enc_foot, b_enc_foot.reshape(1, H))

    agg_bj, agg_jf0 = _segsum_sc(
        [(xb0, sbj, dbj), (xj0, sjf, djf)], zeros_dz)

    xj1, xf1 = _layer0(
        xj0, xf0, agg_bj, agg_jf0,
        W_rel[0, 2], W_root[0, 2], b_rel[0, 2].reshape(1, H),
        W_rel[0, 3], W_root[0, 3], b_rel[0, 3].reshape(1, H))

    (agg_jf1,) = _segsum_sc([(xj1, sjf, djf)], zeros_dz)

    y = _layer1(
        xf1, agg_jf1,
        W_rel[1, 3], W_root[1, 3], b_rel[1, 3].reshape(1, H),
        W_dec, b_dec.reshape(1, 1))
    return y
